# Initial kernel scaffold; baseline (speedup 1.0000x reference)
import jax
import jax.numpy as jnp
from jax.experimental import pallas as pl


def kernel(h, edge_index, W, b, gamma, beta):
    raise NotImplementedError("write your pallas kernel here")


# speedup vs baseline: 2.1044x; 2.1044x over previous
import functools

import jax
import jax.numpy as jnp
from jax import lax
from jax.experimental import pallas as pl
from jax.experimental.pallas import tpu as pltpu
from jax.experimental.pallas import tpu_sc as plsc

N_NODES = 10000
N_EDGES = 320000
D = 128

NC = 2
NS = 16
NW = NC * NS
NPW = 320
NPAD = NW * NPW

ECH = 4000
NCHUNK = N_EDGES // ECH
SCAN_UNROLL = 10
B = 128
MB = 4352


def _mlp_body(h_ref, w_ref, b_ref, g_ref, be_ref, o_ref):
    z = lax.dot_general(h_ref[...], w_ref[...],
                        (((1,), (1,)), ((), ())),
                        preferred_element_type=jnp.float32)
    z = z + b_ref[...]
    mu = jnp.mean(z, axis=1, keepdims=True)
    var = jnp.mean((z - mu) ** 2, axis=1, keepdims=True)
    zn = (z - mu) * lax.rsqrt(var + 1e-5) * g_ref[...] + be_ref[...]
    o_ref[...] = jnp.maximum(zn, 0.0)


def _mlp(h, W, b, gamma, beta):
    blk = 1000
    grid = N_NODES // blk
    return pl.pallas_call(
        _mlp_body,
        grid=(grid,),
        in_specs=[
            pl.BlockSpec((blk, D), lambda i: (i, 0)),
            pl.BlockSpec((D, D), lambda i: (0, 0)),
            pl.BlockSpec((1, D), lambda i: (0, 0)),
            pl.BlockSpec((1, D), lambda i: (0, 0)),
            pl.BlockSpec((1, D), lambda i: (0, 0)),
        ],
        out_specs=pl.BlockSpec((blk, D), lambda i: (i, 0)),
        out_shape=jax.ShapeDtypeStruct((N_NODES, D), jnp.float32),
    )(h, W, b.reshape(1, D), gamma.reshape(1, D), beta.reshape(1, D))


def _seg_body(hh_hbm, src_hbm, dst_hbm, out_hbm,
              src_v, dst_v, msrc, mdst, rows, acc, sem):
    cid = lax.axis_index("c")
    sid = lax.axis_index("s")
    wid = cid * NS + sid
    lo = wid * NPW

    zero = jnp.zeros((16,), jnp.float32)

    def zero_body(r, _):
        for cg in range(D // 16):
            acc[r, pl.ds(cg * 16, 16)] = zero
        return 0

    lax.fori_loop(0, NPW + 1, zero_body, 0)

    def accumulate_batch(off, _):
        pltpu.async_copy(hh_hbm.at[msrc.at[pl.ds(off, B)]], rows, sem).wait()

        def e_body(e, _):
            r = mdst[pl.ds(off + e, 16)][0]
            for cg in range(D // 16):
                sl = pl.ds(cg * 16, 16)
                acc[r, sl] = jnp.maximum(acc[r, sl], rows[e, sl])
            return 0

        lax.fori_loop(0, B, e_body, 0)
        return 0

    def chunk_body(ci, cnt):
        pltpu.sync_copy(src_hbm.at[pl.ds(ci * ECH, ECH)], src_v)
        pltpu.sync_copy(dst_hbm.at[pl.ds(ci * ECH, ECH)], dst_v)

        def scan_body(g, cnt):
            base = g * (SCAN_UNROLL * 16)
            ds_ = []
            ss_ = []
            ms_ = []
            pc_ = []
            for u in range(SCAN_UNROLL):
                d = dst_v[pl.ds(base + u * 16, 16)]
                s = src_v[pl.ds(base + u * 16, 16)]
                m = (d >= lo) & (d < lo + NPW)
                ds_.append(d - lo)
                ss_.append(s)
                ms_.append(m)
                pc_.append(plsc.all_reduce_population_count(m)[0])
            for u in range(SCAN_UNROLL):
                plsc.store_compressed(msrc.at[pl.ds(cnt, 16)], ss_[u],
                                      mask=ms_[u])
                plsc.store_compressed(mdst.at[pl.ds(cnt, 16)], ds_[u],
                                      mask=ms_[u])
                cnt = cnt + pc_[u]
            return cnt

        cnt = lax.fori_loop(0, ECH // (SCAN_UNROLL * 16), scan_body, cnt)

        nb = cnt // B

        def batch_body(bi, _):
            return accumulate_batch(bi * B, _)

        lax.fori_loop(0, nb, batch_body, 0)

        rem = nb * B

        def mv_body(i, _):
            sl_dst = pl.ds(i * 16, 16)
            sl_src = pl.ds(rem + i * 16, 16)
            msrc[sl_dst] = msrc[sl_src]
            mdst[sl_dst] = mdst[sl_src]
            return 0

        lax.fori_loop(0, B // 16, mv_body, 0)
        return cnt - rem

    cnt = lax.fori_loop(0, NCHUNK, chunk_body, 0)

    pad_src = jnp.zeros((16,), jnp.int32)
    pad_dst = jnp.full((16,), NPW, jnp.int32)

    def pad_body(i, _):
        msrc[pl.ds(cnt + i * 16, 16)] = pad_src
        mdst[pl.ds(cnt + i * 16, 16)] = pad_dst
        return 0

    lax.fori_loop(0, B // 16, pad_body, 0)
    accumulate_batch(0, 0)

    pltpu.sync_copy(acc.at[pl.ds(0, NPW)], out_hbm.at[pl.ds(lo, NPW)])


def _segmax(hh, src, dst):
    mesh = plsc.VectorSubcoreMesh(core_axis_name="c", subcore_axis_name="s",
                                  num_cores=NC, num_subcores=NS)
    f = pl.kernel(
        _seg_body,
        out_type=jax.ShapeDtypeStruct((NPAD, D), jnp.float32),
        mesh=mesh,
        compiler_params=pltpu.CompilerParams(needs_layout_passes=False),
        scratch_types=[
            pltpu.VMEM((ECH,), jnp.int32),
            pltpu.VMEM((ECH,), jnp.int32),
            pltpu.VMEM((MB,), jnp.int32),
            pltpu.VMEM((MB,), jnp.int32),
            pltpu.VMEM((B, D), jnp.float32),
            pltpu.VMEM((NPW + 1, D), jnp.float32),
            pltpu.SemaphoreType.DMA,
        ],
    )
    return f(hh, src, dst)


@jax.jit
def kernel(h, edge_index, W, b, gamma, beta):
    hh = _mlp(h, W, b, gamma, beta)
    src = edge_index[0].astype(jnp.int32)
    dst = edge_index[1].astype(jnp.int32)
    h_n = _segmax(hh, src, dst)[:N_NODES]
    return jnp.concatenate([hh, h_n], axis=1)


# speedup vs baseline: 2.5143x; 1.1948x over previous
import functools

import jax
import jax.numpy as jnp
from jax import lax
from jax.experimental import pallas as pl
from jax.experimental.pallas import tpu as pltpu
from jax.experimental.pallas import tpu_sc as plsc

N_NODES = 10000
N_EDGES = 320000
D = 128

NC = 2
NS = 16
NW = NC * NS
NPW = 320
NPAD = NW * NPW

ECH = 4000
NCHUNK = N_EDGES // ECH
SCAN_UNROLL = 10
B = 128
MB = 4352


def _mlp_body(h_ref, w_ref, b_ref, g_ref, be_ref, o_ref):
    z = lax.dot_general(h_ref[...], w_ref[...],
                        (((1,), (1,)), ((), ())),
                        preferred_element_type=jnp.float32)
    z = z + b_ref[...]
    mu = jnp.mean(z, axis=1, keepdims=True)
    var = jnp.mean((z - mu) ** 2, axis=1, keepdims=True)
    zn = (z - mu) * lax.rsqrt(var + 1e-5) * g_ref[...] + be_ref[...]
    o_ref[...] = jnp.maximum(zn, 0.0)


def _mlp(h, W, b, gamma, beta):
    blk = 1000
    grid = N_NODES // blk
    return pl.pallas_call(
        _mlp_body,
        grid=(grid,),
        in_specs=[
            pl.BlockSpec((blk, D), lambda i: (i, 0)),
            pl.BlockSpec((D, D), lambda i: (0, 0)),
            pl.BlockSpec((1, D), lambda i: (0, 0)),
            pl.BlockSpec((1, D), lambda i: (0, 0)),
            pl.BlockSpec((1, D), lambda i: (0, 0)),
        ],
        out_specs=pl.BlockSpec((blk, D), lambda i: (i, 0)),
        out_shape=jax.ShapeDtypeStruct((N_NODES, D), jnp.float32),
    )(h, W, b.reshape(1, D), gamma.reshape(1, D), beta.reshape(1, D))


def _seg_body(hh_hbm, src_hbm, dst_hbm, out_hbm,
              src_v, dst_v, msrc, mdst, rows, acc, sem):
    cid = lax.axis_index("c")
    sid = lax.axis_index("s")
    wid = cid * NS + sid
    lo = wid * NPW

    zero = jnp.zeros((16,), jnp.float32)

    def zero_body(r, _):
        for cg in range(D // 16):
            acc[r, pl.ds(cg * 16, 16)] = zero
        return 0

    lax.fori_loop(0, NPW + 1, zero_body, 0)

    def accumulate_batch(off, _):
        pltpu.async_copy(hh_hbm.at[msrc.at[pl.ds(off, B)]], rows, sem).wait()

        def g_body(g, _):
            dvec = mdst[pl.ds(off + g * 16, 16)]
            e0 = g * 16
            for j in range(16):
                r = dvec[j]
                e = e0 + j
                for cg in range(D // 16):
                    sl = pl.ds(cg * 16, 16)
                    acc[r, sl] = jnp.maximum(acc[r, sl], rows[e, sl])
            return 0

        lax.fori_loop(0, B // 16, g_body, 0)
        return 0

    def chunk_body(ci, cnt):
        pltpu.sync_copy(src_hbm.at[pl.ds(ci * ECH, ECH)], src_v)
        pltpu.sync_copy(dst_hbm.at[pl.ds(ci * ECH, ECH)], dst_v)

        def scan_body(g, cnt):
            base = g * (SCAN_UNROLL * 16)
            ds_ = []
            ss_ = []
            ms_ = []
            pc_ = []
            for u in range(SCAN_UNROLL):
                d = dst_v[pl.ds(base + u * 16, 16)]
                s = src_v[pl.ds(base + u * 16, 16)]
                dml = d - lo
                m = plsc.bitcast(dml, jnp.uint32) < jnp.uint32(NPW)
                ds_.append(dml)
                ss_.append(s)
                ms_.append(m)
                pc_.append(plsc.all_reduce_population_count(m)[0])
            for u in range(SCAN_UNROLL):
                plsc.store_compressed(msrc.at[pl.ds(cnt, 16)], ss_[u],
                                      mask=ms_[u])
                plsc.store_compressed(mdst.at[pl.ds(cnt, 16)], ds_[u],
                                      mask=ms_[u])
                cnt = cnt + pc_[u]
            return cnt

        cnt = lax.fori_loop(0, ECH // (SCAN_UNROLL * 16), scan_body, cnt)

        nb = cnt // B

        def batch_body(bi, _):
            return accumulate_batch(bi * B, _)

        lax.fori_loop(0, nb, batch_body, 0)

        rem = nb * B

        def mv_body(i, _):
            sl_dst = pl.ds(i * 16, 16)
            sl_src = pl.ds(rem + i * 16, 16)
            msrc[sl_dst] = msrc[sl_src]
            mdst[sl_dst] = mdst[sl_src]
            return 0

        lax.fori_loop(0, B // 16, mv_body, 0)
        return cnt - rem

    cnt = lax.fori_loop(0, NCHUNK, chunk_body, 0)

    pad_src = jnp.zeros((16,), jnp.int32)
    pad_dst = jnp.full((16,), NPW, jnp.int32)

    def pad_body(i, _):
        msrc[pl.ds(cnt + i * 16, 16)] = pad_src
        mdst[pl.ds(cnt + i * 16, 16)] = pad_dst
        return 0

    lax.fori_loop(0, B // 16, pad_body, 0)
    accumulate_batch(0, 0)

    pltpu.sync_copy(acc.at[pl.ds(0, NPW)], out_hbm.at[pl.ds(lo, NPW)])


def _segmax(hh, src, dst):
    mesh = plsc.VectorSubcoreMesh(core_axis_name="c", subcore_axis_name="s",
                                  num_cores=NC, num_subcores=NS)
    f = pl.kernel(
        _seg_body,
        out_type=jax.ShapeDtypeStruct((NPAD, D), jnp.float32),
        mesh=mesh,
        compiler_params=pltpu.CompilerParams(needs_layout_passes=False),
        scratch_types=[
            pltpu.VMEM((ECH,), jnp.int32),
            pltpu.VMEM((ECH,), jnp.int32),
            pltpu.VMEM((MB,), jnp.int32),
            pltpu.VMEM((MB,), jnp.int32),
            pltpu.VMEM((B, D), jnp.float32),
            pltpu.VMEM((NPW + 1, D), jnp.float32),
            pltpu.SemaphoreType.DMA,
        ],
    )
    return f(hh, src, dst)


@jax.jit
def kernel(h, edge_index, W, b, gamma, beta):
    hh = _mlp(h, W, b, gamma, beta)
    src = edge_index[0].astype(jnp.int32)
    dst = edge_index[1].astype(jnp.int32)
    h_n = _segmax(hh, src, dst)[:N_NODES]
    return jnp.concatenate([hh, h_n], axis=1)


# speedup vs baseline: 2.9710x; 1.1817x over previous
import functools

import jax
import jax.numpy as jnp
from jax import lax
from jax.experimental import pallas as pl
from jax.experimental.pallas import tpu as pltpu
from jax.experimental.pallas import tpu_sc as plsc

N_NODES = 10000
N_EDGES = 320000
D = 128

NC = 2
NS = 16
NW = NC * NS
NPW = 320
NPAD = NW * NPW

ECH = 4000
NCHUNK = N_EDGES // ECH
SCAN_UNROLL = 10
B = 128
MB = 4352


def _mlp_body(h_ref, w_ref, b_ref, g_ref, be_ref, o_ref):
    z = lax.dot_general(h_ref[...], w_ref[...],
                        (((1,), (1,)), ((), ())),
                        preferred_element_type=jnp.float32)
    z = z + b_ref[...]
    mu = jnp.mean(z, axis=1, keepdims=True)
    var = jnp.mean((z - mu) ** 2, axis=1, keepdims=True)
    zn = (z - mu) * lax.rsqrt(var + 1e-5) * g_ref[...] + be_ref[...]
    o_ref[...] = jnp.maximum(zn, 0.0)


def _mlp(h, W, b, gamma, beta):
    blk = 1000
    grid = N_NODES // blk
    return pl.pallas_call(
        _mlp_body,
        grid=(grid,),
        in_specs=[
            pl.BlockSpec((blk, D), lambda i: (i, 0)),
            pl.BlockSpec((D, D), lambda i: (0, 0)),
            pl.BlockSpec((1, D), lambda i: (0, 0)),
            pl.BlockSpec((1, D), lambda i: (0, 0)),
            pl.BlockSpec((1, D), lambda i: (0, 0)),
        ],
        out_specs=pl.BlockSpec((blk, D), lambda i: (i, 0)),
        out_shape=jax.ShapeDtypeStruct((N_NODES, D), jnp.float32),
    )(h, W, b.reshape(1, D), gamma.reshape(1, D), beta.reshape(1, D))


def _seg_body(hh_hbm, src_hbm, dst_hbm, out_hbm,
              src_v, dst_v, msrc, mdst, rows, acc, sem_e, sem_r):
    cid = lax.axis_index("c")
    sid = lax.axis_index("s")
    wid = cid * NS + sid
    lo = wid * NPW

    zero = jnp.zeros((16,), jnp.float32)

    def zero_body(r, _):
        for cg in range(D // 16):
            acc[r, pl.ds(cg * 16, 16)] = zero
        return 0

    lax.fori_loop(0, NPW + 1, zero_body, 0)

    def fire_chunk(ci, buf):
        sl = pl.ds(buf * ECH, ECH)
        pltpu.async_copy(src_hbm.at[pl.ds(ci * ECH, ECH)], src_v.at[sl],
                         sem_e.at[buf])
        pltpu.async_copy(dst_hbm.at[pl.ds(ci * ECH, ECH)], dst_v.at[sl],
                         sem_e.at[buf])

    def wait_chunk(buf):
        sl = pl.ds(buf * ECH, ECH)
        pltpu.make_async_copy(src_hbm.at[pl.ds(0, ECH)], src_v.at[sl],
                              sem_e.at[buf]).wait()
        pltpu.make_async_copy(dst_hbm.at[pl.ds(0, ECH)], dst_v.at[sl],
                              sem_e.at[buf]).wait()

    def fire_batch(off, buf):
        pltpu.async_copy(hh_hbm.at[msrc.at[pl.ds(off, B)]],
                         rows.at[pl.ds(buf * B, B)], sem_r.at[buf])

    def wait_batch(buf):
        pltpu.make_async_copy(hh_hbm.at[pl.ds(0, B)],
                              rows.at[pl.ds(buf * B, B)],
                              sem_r.at[buf]).wait()

    def accumulate_batch(off, boff):
        def g_body(g, _):
            dvec = mdst[pl.ds(off + g * 16, 16)]
            e0 = boff + g * 16
            for j in range(16):
                r = dvec[j]
                e = e0 + j
                for cg in range(D // 16):
                    sl = pl.ds(cg * 16, 16)
                    acc[r, sl] = jnp.maximum(acc[r, sl], rows[e, sl])
            return 0

        lax.fori_loop(0, B // 16, g_body, 0)

    fire_chunk(0, 0)

    def chunk_body(ci, cnt):
        cur = lax.rem(ci, 2)
        nxt = 1 - cur
        wait_chunk(cur)

        @pl.when(ci + 1 < NCHUNK)
        def _():
            fire_chunk(ci + 1, nxt)

        cbase = cur * ECH

        def scan_body(g, cnt):
            base = cbase + g * (SCAN_UNROLL * 16)
            ds_ = []
            ss_ = []
            ms_ = []
            pc_ = []
            for u in range(SCAN_UNROLL):
                d = dst_v[pl.ds(base + u * 16, 16)]
                s = src_v[pl.ds(base + u * 16, 16)]
                dml = d - lo
                m = plsc.bitcast(dml, jnp.uint32) < jnp.uint32(NPW)
                ds_.append(dml)
                ss_.append(s)
                ms_.append(m)
                pc_.append(plsc.all_reduce_population_count(m)[0])
            for u in range(SCAN_UNROLL):
                plsc.store_compressed(msrc.at[pl.ds(cnt, 16)], ss_[u],
                                      mask=ms_[u])
                plsc.store_compressed(mdst.at[pl.ds(cnt, 16)], ds_[u],
                                      mask=ms_[u])
                cnt = cnt + pc_[u]
            return cnt

        cnt = lax.fori_loop(0, ECH // (SCAN_UNROLL * 16), scan_body, cnt)

        nb = cnt // B

        @pl.when(nb > 0)
        def _():
            fire_batch(0, 0)

        def batch_body(bi, _):
            b_cur = lax.rem(bi, 2)

            @pl.when(bi + 1 < nb)
            def _():
                fire_batch((bi + 1) * B, 1 - b_cur)

            wait_batch(b_cur)
            accumulate_batch(bi * B, b_cur * B)
            return 0

        lax.fori_loop(0, nb, batch_body, 0)

        rem = nb * B

        def mv_body(i, _):
            sl_dst = pl.ds(i * 16, 16)
            sl_src = pl.ds(rem + i * 16, 16)
            msrc[sl_dst] = msrc[sl_src]
            mdst[sl_dst] = mdst[sl_src]
            return 0

        lax.fori_loop(0, B // 16, mv_body, 0)
        return cnt - rem

    cnt = lax.fori_loop(0, NCHUNK, chunk_body, 0)

    pad_src = jnp.zeros((16,), jnp.int32)
    pad_dst = jnp.full((16,), NPW, jnp.int32)

    def pad_body(i, _):
        msrc[pl.ds(cnt + i * 16, 16)] = pad_src
        mdst[pl.ds(cnt + i * 16, 16)] = pad_dst
        return 0

    lax.fori_loop(0, B // 16, pad_body, 0)
    fire_batch(0, 0)
    wait_batch(0)
    accumulate_batch(0, 0)

    pltpu.sync_copy(acc.at[pl.ds(0, NPW)], out_hbm.at[pl.ds(lo, NPW)])


def _segmax(hh, src, dst):
    mesh = plsc.VectorSubcoreMesh(core_axis_name="c", subcore_axis_name="s",
                                  num_cores=NC, num_subcores=NS)
    f = pl.kernel(
        _seg_body,
        out_type=jax.ShapeDtypeStruct((NPAD, D), jnp.float32),
        mesh=mesh,
        compiler_params=pltpu.CompilerParams(needs_layout_passes=False),
        scratch_types=[
            pltpu.VMEM((2 * ECH,), jnp.int32),
            pltpu.VMEM((2 * ECH,), jnp.int32),
            pltpu.VMEM((MB,), jnp.int32),
            pltpu.VMEM((MB,), jnp.int32),
            pltpu.VMEM((2 * B, D), jnp.float32),
            pltpu.VMEM((NPW + 1, D), jnp.float32),
            pltpu.SemaphoreType.DMA((2,)),
            pltpu.SemaphoreType.DMA((2,)),
        ],
    )
    return f(hh, src, dst)


@jax.jit
def kernel(h, edge_index, W, b, gamma, beta):
    hh = _mlp(h, W, b, gamma, beta)
    src = edge_index[0].astype(jnp.int32)
    dst = edge_index[1].astype(jnp.int32)
    h_n = _segmax(hh, src, dst)[:N_NODES]
    return jnp.concatenate([hh, h_n], axis=1)
